# baseline (speedup 1.0000x reference)
import functools

import jax
import jax.numpy as jnp
from jax import lax
from jax.experimental import pallas as pl
from jax.experimental.pallas import tpu as pltpu
from jax.experimental.pallas import tpu_sc as plsc

NC = 2
NS = 16
L = 16
NW = NC * NS

IDX_W = 128
SUB = 8
CHUNK = SUB * IDX_W


@functools.lru_cache(maxsize=None)
def _build(B, F, V, D):
    TOT = B * F
    assert B % CHUNK == 0 and TOT % (NW * CHUNK) == 0
    per_w = TOT // NW
    n_chunks = per_w // CHUNK

    mesh = plsc.VectorSubcoreMesh(core_axis_name="c", subcore_axis_name="s")

    @functools.partial(
        pl.kernel,
        mesh=mesh,
        out_type=jax.ShapeDtypeStruct((TOT, D), jnp.float32),
        scratch_types=[
            pltpu.VMEM((CHUNK,), jnp.int32),
            pltpu.VMEM((SUB, IDX_W), jnp.int32),
            pltpu.VMEM((CHUNK, D), jnp.float32),
            pltpu.SemaphoreType.DMA,
            pltpu.SemaphoreType.DMA,
        ],
        compiler_params=pltpu.CompilerParams(use_tc_tiling_on_sc=False),
    )
    def gather_kernel(xt_hbm, tab_hbm, out_hbm, idx_v, oidx_v, rows_v,
                      gsem, ssem):
        wid = lax.axis_index("s") * NC + lax.axis_index("c")
        lane = lax.iota(jnp.int32, L)

        def chunk_body(c, carry):
            q = pl.multiple_of((wid * n_chunks + c) * CHUNK, 8)
            j = q // B
            b0 = q - j * B
            pltpu.sync_copy(xt_hbm.at[pl.ds(q, CHUNK)], idx_v)
            tab_off = j * V
            dest0 = b0 * F + j
            for t in range(CHUNK // L):
                sl = pl.ds(t * L, L)
                idx_v[sl] = idx_v[sl] + tab_off
                k, l = divmod(t, IDX_W // L)
                oidx_v[k, pl.ds(l * L, L)] = dest0 + (lane + t * L) * F
            gathers = [
                pltpu.async_copy(
                    tab_hbm.at[idx_v.at[pl.ds(k * IDX_W, IDX_W)]],
                    rows_v.at[pl.ds(k * IDX_W, IDX_W)],
                    gsem,
                )
                for k in range(SUB)
            ]
            for cp in gathers:
                cp.wait()
            scatters = [
                pltpu.async_copy(
                    rows_v.at[pl.ds(k * IDX_W, IDX_W)],
                    out_hbm.at[oidx_v.at[k]],
                    ssem,
                )
                for k in range(SUB)
            ]
            for cp in scatters:
                cp.wait()
            return carry

        lax.fori_loop(0, n_chunks, chunk_body, None)

    return gather_kernel


def kernel(x_cat, tables):
    B, F = x_cat.shape
    _, V, D = tables.shape
    xt_flat = x_cat.T.reshape(-1)
    tab = tables.reshape(F * V, D)
    out = _build(B, F, V, D)(xt_flat, tab)
    return out.reshape(B, F * D)
